# baseline (speedup 1.0000x reference)
import jax
import jax.numpy as jnp
from jax import lax
from jax.experimental import pallas as pl
from jax.experimental.pallas import tpu as pltpu
from jax.experimental.pallas import tpu_sc as plsc
import functools

B = 128
VOCAB = 100000
L = 16
NCHUNK = B // L


@functools.partial(
    pl.kernel,
    out_type=jax.ShapeDtypeStruct((L,), jnp.float32),
    mesh=plsc.VectorSubcoreMesh(core_axis_name="c", subcore_axis_name="s"),
    compiler_params=pltpu.CompilerParams(
        needs_layout_passes=False, use_tc_tiling_on_sc=False
    ),
    scratch_types=[
        pltpu.VMEM((B,), jnp.int32),
        pltpu.VMEM((B, L), jnp.float32),
        pltpu.VMEM((L,), jnp.float32),
        pltpu.SemaphoreType.DMA,
    ],
)
def _logit_loss_sc(logits_hbm, labels_hbm, out_hbm, lab_v, win_v, res_v, sem):
    tile0 = jnp.logical_and(lax.axis_index("c") == 0, lax.axis_index("s") == 0)

    @pl.when(tile0)
    def _():
        pltpu.sync_copy(labels_hbm, lab_v)
        copies = []
        for c in range(NCHUNK):
            starts = (lab_v[pl.ds(c * L, L)] >> 4) << 4
            for j in range(L):
                i = c * L + j
                start = pl.multiple_of(starts[j], L)
                copies.append(
                    pltpu.async_copy(
                        logits_hbm.at[i, pl.ds(start, L)], win_v.at[i], sem
                    )
                )
        for cp in copies:
            cp.wait()

        lane = lax.iota(jnp.int32, L)
        acc = None
        for c in range(NCHUNK):
            rows = c * L + lane
            cols = lab_v[pl.ds(c * L, L)] & (L - 1)
            g = plsc.load_gather(win_v, [rows, cols])
            acc = g if acc is None else acc + g
        for shift in (8, 4, 2, 1):
            res_v[...] = acc
            acc = acc + plsc.load_gather(res_v, [lane ^ shift])
        res_v[...] = acc
        pltpu.sync_copy(res_v, out_hbm)


def kernel(logits, labels):
    out = _logit_loss_sc(logits, labels.astype(jnp.int32))
    return out[0]


# speedup vs baseline: 2.0423x; 2.0423x over previous
import jax
import jax.numpy as jnp
from jax import lax
from jax.experimental import pallas as pl
from jax.experimental.pallas import tpu as pltpu
from jax.experimental.pallas import tpu_sc as plsc
import functools

B = 128
VOCAB = 100000
L = 16
NS = 16
RPS = B // NS
SUB = 8
LANE = 128


@functools.partial(
    pl.kernel,
    out_type=jax.ShapeDtypeStruct((L,), jnp.float32),
    mesh=plsc.VectorSubcoreMesh(core_axis_name="c", subcore_axis_name="s"),
    compiler_params=pltpu.CompilerParams(needs_layout_passes=False),
    scratch_types=[
        pltpu.VMEM((L,), jnp.int32),
        pltpu.VMEM((RPS, SUB, LANE), jnp.float32),
        pltpu.VMEM((L,), jnp.float32),
        pltpu.VMEM((NS * L,), jnp.float32),
        pltpu.VMEM_SHARED((NS * L,), jnp.float32),
        pltpu.SemaphoreType.DMA,
    ],
)
def _logit_loss_sc(
    logits_hbm, labels_hbm, out_hbm, lab_v, win_v, res_v, red_v, shared, sem
):
    cid = lax.axis_index("c")
    sid = lax.axis_index("s")

    @pl.when(cid == 0)
    def _():
        base = pl.multiple_of(sid * RPS, RPS)
        pltpu.sync_copy(labels_hbm.at[pl.ds(base, L)], lab_v)
        labs = lab_v[...]
        col0s = (labs >> 7) << 7
        copies = []
        for j in range(RPS):
            col0 = pl.multiple_of(col0s[j], LANE)
            copies.append(
                pltpu.async_copy(
                    logits_hbm.at[pl.ds(base, SUB), pl.ds(col0, LANE)],
                    win_v.at[j],
                    sem,
                )
            )
        for cp in copies:
            cp.wait()

        lane = lax.iota(jnp.int32, L)
        j_idx = lane & (RPS - 1)
        g = plsc.load_gather(win_v, [j_idx, j_idx, labs & (LANE - 1)])
        part = jnp.where(lane < RPS, g, 0.0)
        res_v[...] = part
        pltpu.sync_copy(res_v, shared.at[pl.ds(pl.multiple_of(sid * L, L), L)])
        plsc.subcore_barrier()

        @pl.when(sid == 0)
        def _():
            pltpu.sync_copy(shared, red_v)
            acc = red_v[pl.ds(0, L)]
            for k in range(1, NS):
                acc = acc + red_v[pl.ds(k * L, L)]
            for shift in (8, 4, 2, 1):
                res_v[...] = acc
                acc = acc + plsc.load_gather(res_v, [lane ^ shift])
            res_v[...] = acc
            pltpu.sync_copy(res_v, out_hbm)


def kernel(logits, labels):
    labels_pad = jnp.zeros((2 * B,), jnp.int32).at[:B].set(labels.astype(jnp.int32))
    out = _logit_loss_sc(logits, labels_pad)
    return out[0]


# speedup vs baseline: 2.0851x; 1.0210x over previous
import jax
import jax.numpy as jnp
from jax import lax
from jax.experimental import pallas as pl
from jax.experimental.pallas import tpu as pltpu
from jax.experimental.pallas import tpu_sc as plsc
import functools

B = 128
VOCAB = 100000
L = 16
NS = 16
RPS = B // NS
SUB = 8
LANE = 128


@functools.partial(
    pl.kernel,
    out_type=jax.ShapeDtypeStruct((L,), jnp.float32),
    mesh=plsc.VectorSubcoreMesh(
        core_axis_name="c", subcore_axis_name="s", num_cores=1
    ),
    compiler_params=pltpu.CompilerParams(needs_layout_passes=False),
    scratch_types=[
        pltpu.VMEM((L,), jnp.int32),
        pltpu.VMEM((RPS, SUB, LANE), jnp.float32),
        pltpu.VMEM((L,), jnp.float32),
        pltpu.VMEM((NS * L,), jnp.float32),
        pltpu.VMEM_SHARED((NS * L,), jnp.float32),
        pltpu.SemaphoreType.DMA,
    ],
)
def _logit_loss_sc(
    logits_hbm, labels_hbm, out_hbm, lab_v, win_v, res_v, red_v, shared, sem
):
    cid = lax.axis_index("c")
    sid = lax.axis_index("s")

    @pl.when(cid == 0)
    def _():
        base = pl.multiple_of(sid * RPS, RPS)
        pltpu.sync_copy(labels_hbm.at[pl.ds(base, L)], lab_v)
        labs = lab_v[...]
        col0s = (labs >> 7) << 7
        copies = []
        for j in range(RPS):
            col0 = pl.multiple_of(col0s[j], LANE)
            copies.append(
                pltpu.async_copy(
                    logits_hbm.at[pl.ds(base, SUB), pl.ds(col0, LANE)],
                    win_v.at[j],
                    sem,
                )
            )
        for cp in copies:
            cp.wait()

        lane = lax.iota(jnp.int32, L)
        j_idx = lane & (RPS - 1)
        g = plsc.load_gather(win_v, [j_idx, j_idx, labs & (LANE - 1)])
        part = jnp.where(lane < RPS, g, 0.0)
        res_v[...] = part
        pltpu.sync_copy(res_v, shared.at[pl.ds(pl.multiple_of(sid * L, L), L)])
        plsc.subcore_barrier()

        @pl.when(sid == 0)
        def _():
            pltpu.sync_copy(shared, red_v)
            acc = red_v[pl.ds(0, L)]
            for k in range(1, NS):
                acc = acc + red_v[pl.ds(k * L, L)]
            for shift in (8, 4, 2, 1):
                res_v[...] = acc
                acc = acc + plsc.load_gather(res_v, [lane ^ shift])
            res_v[...] = acc
            pltpu.sync_copy(res_v, out_hbm)


def kernel(logits, labels):
    labels_pad = jnp.zeros((2 * B,), jnp.int32).at[:B].set(labels.astype(jnp.int32))
    out = _logit_loss_sc(logits, labels_pad)
    return out[0]


# speedup vs baseline: 2.1025x; 1.0084x over previous
import jax
import jax.numpy as jnp
from jax import lax
from jax.experimental import pallas as pl
from jax.experimental.pallas import tpu as pltpu
from jax.experimental.pallas import tpu_sc as plsc
import functools

B = 128
VOCAB = 100000
L = 16
NS = 16
RPS = B // NS
SUB = 8
LANE = 128


@functools.partial(
    pl.kernel,
    out_type=jax.ShapeDtypeStruct((L,), jnp.float32),
    mesh=plsc.VectorSubcoreMesh(
        core_axis_name="c", subcore_axis_name="s", num_cores=1
    ),
    compiler_params=pltpu.CompilerParams(
        needs_layout_passes=False, use_tc_tiling_on_sc=True
    ),
    scratch_types=[
        pltpu.VMEM((L,), jnp.int32),
        pltpu.VMEM((RPS, SUB, LANE), jnp.float32),
        pltpu.VMEM((L,), jnp.float32),
        pltpu.VMEM((NS * L,), jnp.float32),
        pltpu.VMEM_SHARED((NS * L,), jnp.float32),
        pltpu.SemaphoreType.DMA,
    ],
)
def _logit_loss_sc(
    logits_hbm, labels_hbm, out_hbm, lab_v, win_v, res_v, red_v, shared, sem
):
    cid = lax.axis_index("c")
    sid = lax.axis_index("s")

    @pl.when(cid == 0)
    def _():
        base = pl.multiple_of(sid * RPS, RPS)
        pltpu.sync_copy(labels_hbm.at[pl.ds(base, L)], lab_v)
        labs = lab_v[...]
        col0s = (labs >> 7) << 7
        copies = []
        for j in range(RPS):
            col0 = pl.multiple_of(col0s[j], LANE)
            copies.append(
                pltpu.async_copy(
                    logits_hbm.at[pl.ds(base, SUB), pl.ds(col0, LANE)],
                    win_v.at[j],
                    sem,
                )
            )
        for cp in copies:
            cp.wait()

        lane = lax.iota(jnp.int32, L)
        j_idx = lane & (RPS - 1)
        g = plsc.load_gather(win_v, [j_idx, j_idx, labs & (LANE - 1)])
        part = jnp.where(lane < RPS, g, 0.0)
        res_v[...] = part
        pltpu.sync_copy(res_v, shared.at[pl.ds(pl.multiple_of(sid * L, L), L)])
        plsc.subcore_barrier()

        @pl.when(sid == 0)
        def _():
            pltpu.sync_copy(shared, red_v)
            acc = red_v[pl.ds(0, L)]
            for k in range(1, NS):
                acc = acc + red_v[pl.ds(k * L, L)]
            for shift in (8, 4, 2, 1):
                res_v[...] = acc
                acc = acc + plsc.load_gather(res_v, [lane ^ shift])
            res_v[...] = acc
            pltpu.sync_copy(res_v, out_hbm)


def kernel(logits, labels):
    labels_pad = jnp.zeros((2 * B,), jnp.int32).at[:B].set(labels.astype(jnp.int32))
    out = _logit_loss_sc(logits, labels_pad)
    return out[0]


# speedup vs baseline: 6.7675x; 3.2188x over previous
import jax
import jax.numpy as jnp
from jax import lax
from jax.experimental import pallas as pl
from jax.experimental.pallas import tpu as pltpu
from jax.experimental.pallas import tpu_sc as plsc
import functools

B = 128
VOCAB = 100000
L = 16
NS = 16
CPS = B // NS
SUB = 8
LANE = 128


@functools.partial(
    pl.kernel,
    out_type=jax.ShapeDtypeStruct((L,), jnp.float32),
    mesh=plsc.VectorSubcoreMesh(
        core_axis_name="c", subcore_axis_name="s", num_cores=1
    ),
    compiler_params=pltpu.CompilerParams(needs_layout_passes=False),
    scratch_types=[
        pltpu.VMEM((L,), jnp.int32),
        pltpu.VMEM((CPS, SUB, LANE), jnp.float32),
        pltpu.VMEM((L,), jnp.float32),
        pltpu.VMEM((NS * L,), jnp.float32),
        pltpu.VMEM_SHARED((NS * L,), jnp.float32),
        pltpu.SemaphoreType.DMA,
    ],
)
def _logit_loss_sc(
    logits_t_hbm, labels_hbm, out_hbm, lab_v, win_v, res_v, red_v, shared, sem
):
    sid = lax.axis_index("s")
    lane = lax.iota(jnp.int32, L)

    chunk = pl.multiple_of((sid >> 1) << 4, L)
    pltpu.sync_copy(labels_hbm.at[pl.ds(chunk, L)], lab_v)
    labs = plsc.load_gather(lab_v, [((sid & 1) << 3) + (lane & (CPS - 1))])
    row0s = (labs >> 3) << 3
    copies = []
    for j in range(CPS):
        row0 = pl.multiple_of(row0s[j], SUB)
        copies.append(
            pltpu.async_copy(
                logits_t_hbm.at[pl.ds(row0, SUB), pl.ds(0, LANE)],
                win_v.at[j],
                sem,
            )
        )
    for cp in copies:
        cp.wait()

    cols = (sid << 3) + (lane & (CPS - 1))
    g = plsc.load_gather(win_v, [lane & (CPS - 1), labs & (SUB - 1), cols])
    part = jnp.where(lane < CPS, g, 0.0)
    res_v[...] = part
    pltpu.sync_copy(res_v, shared.at[pl.ds(pl.multiple_of(sid * L, L), L)])
    plsc.subcore_barrier()

    @pl.when(sid == 0)
    def _():
        pltpu.sync_copy(shared, red_v)
        acc = red_v[pl.ds(0, L)]
        for k in range(1, NS):
            acc = acc + red_v[pl.ds(k * L, L)]
        for shift in (8, 4, 2, 1):
            res_v[...] = acc
            acc = acc + plsc.load_gather(res_v, [lane ^ shift])
        res_v[...] = acc
        pltpu.sync_copy(res_v, out_hbm)


def kernel(logits, labels):
    out = _logit_loss_sc(logits.T, labels.astype(jnp.int32))
    return out[0]


# speedup vs baseline: 6.7996x; 1.0047x over previous
import jax
import jax.numpy as jnp
from jax import lax
from jax.experimental import pallas as pl
from jax.experimental.pallas import tpu as pltpu
from jax.experimental.pallas import tpu_sc as plsc
import functools

B = 128
VOCAB = 100000
L = 16
NCHUNK = B // L


@functools.partial(
    pl.kernel,
    out_type=jax.ShapeDtypeStruct((L,), jnp.float32),
    mesh=plsc.VectorSubcoreMesh(
        core_axis_name="c", subcore_axis_name="s", num_cores=1
    ),
    compiler_params=pltpu.CompilerParams(needs_layout_passes=False),
    scratch_types=[
        pltpu.VMEM((B,), jnp.int32),
        pltpu.VMEM((B, B), jnp.float32),
        pltpu.VMEM((L,), jnp.float32),
        pltpu.SemaphoreType.DMA,
    ],
)
def _logit_loss_sc(logits_t_hbm, labels_hbm, out_hbm, idx_v, rows_v, res_v, sem):
    sid = lax.axis_index("s")

    @pl.when(sid == 0)
    def _():
        pltpu.sync_copy(labels_hbm, idx_v)
        pltpu.async_copy(logits_t_hbm.at[idx_v], rows_v, sem).wait()
        lane = lax.iota(jnp.int32, L)
        acc = None
        for c in range(NCHUNK):
            diag = c * L + lane
            g = plsc.load_gather(rows_v, [diag, diag])
            acc = g if acc is None else acc + g
        for shift in (8, 4, 2, 1):
            res_v[...] = acc
            acc = acc + plsc.load_gather(res_v, [lane ^ shift])
        res_v[...] = acc
        pltpu.sync_copy(res_v, out_hbm)


def kernel(logits, labels):
    out = _logit_loss_sc(logits.T, labels.astype(jnp.int32))
    return out[0]


# speedup vs baseline: 6.9176x; 1.0174x over previous
import jax
import jax.numpy as jnp
from jax import lax
from jax.experimental import pallas as pl
from jax.experimental.pallas import tpu as pltpu
from jax.experimental.pallas import tpu_sc as plsc
import functools

B = 128
VOCAB = 100000
L = 16
NCHUNK = B // L


@functools.partial(
    pl.kernel,
    out_type=jax.ShapeDtypeStruct((L,), jnp.float32),
    mesh=plsc.VectorSubcoreMesh(
        core_axis_name="c", subcore_axis_name="s", num_cores=1
    ),
    compiler_params=pltpu.CompilerParams(needs_layout_passes=False),
    scratch_types=[
        pltpu.VMEM((B,), jnp.int32),
        pltpu.VMEM((B,), jnp.int32),
        pltpu.VMEM((B,), jnp.float32),
        pltpu.VMEM((L,), jnp.float32),
        pltpu.SemaphoreType.DMA,
    ],
)
def _logit_loss_sc(flat_hbm, labels_hbm, out_hbm, lab_v, idx_v, val_v, res_v, sem):
    sid = lax.axis_index("s")

    @pl.when(sid == 0)
    def _():
        pltpu.sync_copy(labels_hbm, lab_v)
        lane = lax.iota(jnp.int32, L)
        for c in range(NCHUNK):
            labs = lab_v[pl.ds(c * L, L)]
            idx_v[pl.ds(c * L, L)] = (labs << 7) + c * L + lane
        pltpu.async_copy(flat_hbm.at[idx_v], val_v, sem).wait()
        acc = val_v[pl.ds(0, L)]
        for c in range(1, NCHUNK):
            acc = acc + val_v[pl.ds(c * L, L)]
        for shift in (8, 4, 2, 1):
            res_v[...] = acc
            acc = acc + plsc.load_gather(res_v, [lane ^ shift])
        res_v[...] = acc
        pltpu.sync_copy(res_v, out_hbm)


def kernel(logits, labels):
    flat = logits.T.reshape(-1)
    out = _logit_loss_sc(flat, labels.astype(jnp.int32))
    return out[0]


# speedup vs baseline: 6.9198x; 1.0003x over previous
import jax
import jax.numpy as jnp
from jax import lax
from jax.experimental import pallas as pl
from jax.experimental.pallas import tpu as pltpu
from jax.experimental.pallas import tpu_sc as plsc
import functools

B = 128
VOCAB = 100000
L = 16
NCHUNK = B // L


@functools.partial(
    pl.kernel,
    out_type=jax.ShapeDtypeStruct((L,), jnp.float32),
    mesh=plsc.VectorSubcoreMesh(
        core_axis_name="c", subcore_axis_name="s", num_cores=1
    ),
    compiler_params=pltpu.CompilerParams(
        needs_layout_passes=False, skip_device_barrier=True
    ),
    scratch_types=[
        pltpu.VMEM((B,), jnp.int32),
        pltpu.VMEM((B,), jnp.int32),
        pltpu.VMEM((B,), jnp.float32),
        pltpu.VMEM((L,), jnp.float32),
        pltpu.SemaphoreType.DMA,
    ],
)
def _logit_loss_sc(flat_hbm, labels_hbm, out_hbm, lab_v, idx_v, val_v, res_v, sem):
    sid = lax.axis_index("s")

    @pl.when(sid == 0)
    def _():
        pltpu.sync_copy(labels_hbm, lab_v)
        lane = lax.iota(jnp.int32, L)
        for c in range(NCHUNK):
            labs = lab_v[pl.ds(c * L, L)]
            idx_v[pl.ds(c * L, L)] = (labs << 7) + c * L + lane
        pltpu.async_copy(flat_hbm.at[idx_v], val_v, sem).wait()
        acc = val_v[pl.ds(0, L)]
        for c in range(1, NCHUNK):
            acc = acc + val_v[pl.ds(c * L, L)]
        for shift in (8, 4, 2, 1):
            res_v[...] = acc
            acc = acc + plsc.load_gather(res_v, [lane ^ shift])
        res_v[...] = acc
        pltpu.sync_copy(res_v, out_hbm)


def kernel(logits, labels):
    flat = logits.T.reshape(-1)
    out = _logit_loss_sc(flat, labels.astype(jnp.int32))
    return out[0]


# speedup vs baseline: 6.9446x; 1.0036x over previous
import jax
import jax.numpy as jnp
from jax import lax
from jax.experimental import pallas as pl
from jax.experimental.pallas import tpu as pltpu
from jax.experimental.pallas import tpu_sc as plsc
import functools

B = 128
VOCAB = 100000
L = 16
NCHUNK = B // L


@functools.partial(
    pl.kernel,
    out_type=jax.ShapeDtypeStruct((L,), jnp.float32),
    mesh=plsc.VectorSubcoreMesh(
        core_axis_name="c", subcore_axis_name="s", num_cores=1
    ),
    compiler_params=pltpu.CompilerParams(needs_layout_passes=False),
    scratch_types=[
        pltpu.VMEM((B,), jnp.int32),
        pltpu.VMEM((B,), jnp.int32),
        pltpu.VMEM((B,), jnp.float32),
        pltpu.VMEM((L,), jnp.float32),
        pltpu.SemaphoreType.DMA,
    ],
)
def _logit_loss_sc(flat_hbm, labels_hbm, out_hbm, lab_v, idx_v, val_v, res_v, sem):
    sid = lax.axis_index("s")

    @pl.when(sid == 0)
    def _():
        pltpu.sync_copy(labels_hbm, lab_v)
        lane = lax.iota(jnp.int32, L)
        for c in range(NCHUNK):
            labs = lab_v[pl.ds(c * L, L)]
            idx_v[pl.ds(c * L, L)] = (labs << 7) + c * L + lane
        pltpu.async_copy(flat_hbm.at[idx_v], val_v, sem).wait()
        acc = val_v[pl.ds(0, L)]
        for c in range(1, NCHUNK):
            acc = acc + val_v[pl.ds(c * L, L)]
        for shift in (8, 4, 2, 1):
            res_v[...] = acc
            acc = acc + plsc.load_gather(res_v, [lane ^ shift])
        res_v[...] = acc
        pltpu.sync_copy(res_v, out_hbm)


def kernel(logits, labels):
    flat = logits.T.reshape(-1)
    out = _logit_loss_sc(flat, labels.astype(jnp.int32))
    return out[0]
